# baseline (device time: 90586 ns/iter reference)
import jax
import jax.numpy as jnp
from jax import lax
from jax.experimental import pallas as pl
from jax.experimental.pallas import tpu as pltpu

N_DEV = 4
N_SUB = 4


def kernel(A, B):
    m, k = A.shape
    _, n = B.shape
    mc = m // N_DEV
    ms = mc // N_SUB
    nh = n // 2

    def kernel_body(a_hbm, b_hbm, out_ref, astage, bstage, pbuf,
                    rs_cw, rs_ccw, bbf, obuf, agr, cp_sems, out_sems,
                    s_cw, r_cw, s_ccw, r_ccw,
                    ag_s_cw, ag_r_cw, ag_s_ccw, ag_r_ccw):
        my = lax.axis_index("i")
        left = lax.rem(my + (N_DEV - 1), N_DEV)
        right = lax.rem(my + 1, N_DEV)

        b_cp = pltpu.make_async_copy(b_hbm, bstage, cp_sems.at[0])
        b_cp.start()
        c_cw1 = lax.rem(my + 3, N_DEV)
        c_ccw1 = lax.rem(my + 1, N_DEV)
        c2 = lax.rem(my + 2, N_DEV)
        fetch_order = [my, c_cw1, c_ccw1, c2]
        a_cps = []
        for i, c in enumerate(fetch_order):
            cp = pltpu.make_async_copy(
                a_hbm.at[pl.ds(c * mc, mc), :],
                astage.at[pl.ds(c * mc, mc), :],
                cp_sems.at[1 + i],
            )
            cp.start()
            a_cps.append(cp)

        barrier_sem = pltpu.get_barrier_semaphore()
        for nbr in (left, right):
            pl.semaphore_signal(
                barrier_sem, inc=1,
                device_id=(nbr,), device_id_type=pl.DeviceIdType.MESH,
            )
        pl.semaphore_wait(barrier_sem, 2)

        def compute_half(c, h):
            ac = astage[pl.ds(c * mc, mc), :].astype(jnp.bfloat16)
            pc = jnp.dot(ac, bbf[:, pl.ds(h * nh, nh)],
                         preferred_element_type=jnp.float32)
            pbuf[pl.ds(c * mc, mc), pl.ds(h * nh, nh)] = pc.astype(
                jnp.bfloat16)

        def rs_send(dirn, s, j, c):
            col0 = 0 if dirn == 0 else nh
            buf = rs_cw if dirn == 0 else rs_ccw
            ssem = s_cw if dirn == 0 else s_ccw
            rsem = r_cw if dirn == 0 else r_ccw
            tgt = right if dirn == 0 else left
            d = pltpu.make_async_remote_copy(
                src_ref=pbuf.at[pl.ds(c * mc + j * ms, ms),
                                pl.ds(col0, nh)],
                dst_ref=buf.at[s, pl.ds(j * ms, ms), :],
                send_sem=ssem.at[s * N_SUB + j],
                recv_sem=rsem.at[s * N_SUB + j],
                device_id=(tgt,), device_id_type=pl.DeviceIdType.MESH,
            )
            d.start()
            return d

        def ag_send(dirn, s, j):
            ssem = ag_s_cw if dirn == 0 else ag_s_ccw
            rsem = ag_r_cw if dirn == 0 else ag_r_ccw
            tgt = right if dirn == 0 else left
            src = (obuf.at[dirn, pl.ds(j * ms, ms), :] if s == 0
                   else agr.at[dirn, s - 1, pl.ds(j * ms, ms), :])
            d = pltpu.make_async_remote_copy(
                src_ref=src,
                dst_ref=agr.at[dirn, s, pl.ds(j * ms, ms), :],
                send_sem=ssem.at[s * N_SUB + j],
                recv_sem=rsem.at[s * N_SUB + j],
                device_id=(tgt,), device_id_type=pl.DeviceIdType.MESH,
            )
            d.start()
            return d

        out_cps = []

        def out_store(src, c, col0):
            cp = pltpu.make_async_copy(
                src, out_ref.at[pl.ds(c * mc, mc), pl.ds(col0, nh)],
                out_sems.at[len(out_cps)],
            )
            cp.start()
            out_cps.append(cp)

        cw = {}
        ccw = {}
        b_cp.wait()
        bbf[...] = bstage[...].astype(jnp.bfloat16)
        a_cps[0].wait()
        compute_half(my, 0)
        for j in range(N_SUB):
            cw[(0, j)] = rs_send(0, 0, j, my)
        compute_half(my, 1)
        for j in range(N_SUB):
            ccw[(0, j)] = rs_send(1, 0, j, my)

        a_cps[1].wait()
        compute_half(c_cw1, 0)
        a_cps[2].wait()
        compute_half(c_ccw1, 1)

        for s in (1, 2):
            c_cw = lax.rem(my - s + N_DEV, N_DEV)
            c_ccw = lax.rem(my + s, N_DEV)
            for j in range(N_SUB):
                cw[(s - 1, j)].wait_recv()
                row = pl.ds(c_cw * mc + j * ms, ms)
                pbuf[row, pl.ds(0, nh)] = (
                    rs_cw[s - 1, pl.ds(j * ms, ms), :]
                    + pbuf[row, pl.ds(0, nh)])
                cw[(s, j)] = rs_send(0, s, j, c_cw)

                ccw[(s - 1, j)].wait_recv()
                row = pl.ds(c_ccw * mc + j * ms, ms)
                pbuf[row, pl.ds(nh, nh)] = (
                    rs_ccw[s - 1, pl.ds(j * ms, ms), :]
                    + pbuf[row, pl.ds(nh, nh)])
                ccw[(s, j)] = rs_send(1, s, j, c_ccw)
            if s == 1:
                a_cps[3].wait()
                compute_half(c2, 0)
                compute_half(c2, 1)

        o_cw = lax.rem(my + 1, N_DEV)
        o_ccw = lax.rem(my + 3, N_DEV)
        compute_half(o_cw, 0)
        compute_half(o_ccw, 1)

        ag_cw = {}
        ag_ccw = {}

        for j in range(N_SUB):
            cw[(2, j)].wait_recv()
        z = (rs_cw[2].astype(jnp.float32)
             + pbuf[pl.ds(o_cw * mc, mc), pl.ds(0, nh)].astype(jnp.float32))
        obuf[0] = (z * jax.nn.sigmoid(z)).astype(jnp.bfloat16)
        for j in range(N_SUB):
            ag_cw[(0, j)] = ag_send(0, 0, j)
        out_store(obuf.at[0], o_cw, 0)

        for j in range(N_SUB):
            ccw[(2, j)].wait_recv()
        z = (rs_ccw[2].astype(jnp.float32)
             + pbuf[pl.ds(o_ccw * mc, mc), pl.ds(nh, nh)].astype(
                 jnp.float32))
        obuf[1] = (z * jax.nn.sigmoid(z)).astype(jnp.bfloat16)
        for j in range(N_SUB):
            ag_ccw[(0, j)] = ag_send(1, 0, j)
        out_store(obuf.at[1], o_ccw, nh)

        for s in (1, 2):
            for j in range(N_SUB):
                ag_cw[(s - 1, j)].wait_recv()
                ag_cw[(s, j)] = ag_send(0, s, j)
                ag_ccw[(s - 1, j)].wait_recv()
                ag_ccw[(s, j)] = ag_send(1, s, j)
            out_store(agr.at[0, s - 1],
                      lax.rem(my - (s - 1) + N_DEV, N_DEV), 0)
            out_store(agr.at[1, s - 1], lax.rem(my + (s - 1), N_DEV), nh)
        for j in range(N_SUB):
            ag_cw[(2, j)].wait_recv()
            ag_ccw[(2, j)].wait_recv()
        out_store(agr.at[0, 2], lax.rem(my - 2 + N_DEV, N_DEV), 0)
        out_store(agr.at[1, 2], lax.rem(my + 2, N_DEV), nh)

        for cp in out_cps:
            cp.wait()
        for d in (list(cw.values()) + list(ccw.values())
                  + list(ag_cw.values()) + list(ag_ccw.values())):
            d.wait_send()

    n_sem = (N_DEV - 1) * N_SUB
    return pl.pallas_call(
        kernel_body,
        out_shape=jax.ShapeDtypeStruct((m, n), jnp.bfloat16),
        in_specs=[
            pl.BlockSpec(memory_space=pl.ANY),
            pl.BlockSpec(memory_space=pl.ANY),
        ],
        out_specs=pl.BlockSpec(memory_space=pl.ANY),
        scratch_shapes=[
            pltpu.VMEM((m, k), jnp.float32),
            pltpu.VMEM((k, n), jnp.float32),
            pltpu.VMEM((m, n), jnp.bfloat16),
            pltpu.VMEM((N_DEV - 1, mc, nh), jnp.bfloat16),
            pltpu.VMEM((N_DEV - 1, mc, nh), jnp.bfloat16),
            pltpu.VMEM((k, n), jnp.bfloat16),
            pltpu.VMEM((2, mc, nh), jnp.bfloat16),
            pltpu.VMEM((2, N_DEV - 1, mc, nh), jnp.bfloat16),
            pltpu.SemaphoreType.DMA((1 + N_DEV,)),
            pltpu.SemaphoreType.DMA((8,)),
            pltpu.SemaphoreType.DMA((n_sem,)),
            pltpu.SemaphoreType.DMA((n_sem,)),
            pltpu.SemaphoreType.DMA((n_sem,)),
            pltpu.SemaphoreType.DMA((n_sem,)),
            pltpu.SemaphoreType.DMA((n_sem,)),
            pltpu.SemaphoreType.DMA((n_sem,)),
            pltpu.SemaphoreType.DMA((n_sem,)),
            pltpu.SemaphoreType.DMA((n_sem,)),
        ],
        compiler_params=pltpu.CompilerParams(
            collective_id=0,
            vmem_limit_bytes=128 * 1024 * 1024,
        ),
    )(A, B)


# device time: 87307 ns/iter; 1.0376x vs baseline; 1.0376x over previous
import jax
import jax.numpy as jnp
from jax import lax
from jax.experimental import pallas as pl
from jax.experimental.pallas import tpu as pltpu

N_DEV = 4
N_SUB = 2


def kernel(A, B):
    m, k = A.shape
    _, n = B.shape
    mc = m // N_DEV
    ms = mc // N_SUB
    nh = n // 2

    def kernel_body(a_hbm, b_hbm, out_ref, astage, bstage, pbuf,
                    rs_cw, rs_ccw, bbf, obuf, agr, cp_sems, out_sems,
                    s_cw, r_cw, s_ccw, r_ccw,
                    ag_s_cw, ag_r_cw, ag_s_ccw, ag_r_ccw):
        my = lax.axis_index("i")
        left = lax.rem(my + (N_DEV - 1), N_DEV)
        right = lax.rem(my + 1, N_DEV)

        b_cps = []
        for h in range(2):
            cp = pltpu.make_async_copy(
                b_hbm.at[:, pl.ds(h * nh, nh)],
                bstage.at[:, pl.ds(h * nh, nh)],
                cp_sems.at[5 + h],
            )
            cp.start()
            b_cps.append(cp)
        c_cw1 = lax.rem(my + 3, N_DEV)
        c_ccw1 = lax.rem(my + 1, N_DEV)
        c2 = lax.rem(my + 2, N_DEV)
        fetch_order = [my, c_cw1, c_ccw1, c2]
        a_cps = []
        for i, c in enumerate(fetch_order):
            cp = pltpu.make_async_copy(
                a_hbm.at[pl.ds(c * mc, mc), :],
                astage.at[pl.ds(c * mc, mc), :],
                cp_sems.at[1 + i],
            )
            cp.start()
            a_cps.append(cp)

        barrier_sem = pltpu.get_barrier_semaphore()
        for nbr in (left, right):
            pl.semaphore_signal(
                barrier_sem, inc=1,
                device_id=(nbr,), device_id_type=pl.DeviceIdType.MESH,
            )
        pl.semaphore_wait(barrier_sem, 2)

        def compute_half(c, h):
            ac = astage[pl.ds(c * mc, mc), :].astype(jnp.bfloat16)
            pc = jnp.dot(ac, bbf[:, pl.ds(h * nh, nh)],
                         preferred_element_type=jnp.float32)
            pbuf[pl.ds(c * mc, mc), pl.ds(h * nh, nh)] = pc.astype(
                jnp.bfloat16)

        def rs_send(dirn, s, j, c):
            col0 = 0 if dirn == 0 else nh
            buf = rs_cw if dirn == 0 else rs_ccw
            ssem = s_cw if dirn == 0 else s_ccw
            rsem = r_cw if dirn == 0 else r_ccw
            tgt = right if dirn == 0 else left
            d = pltpu.make_async_remote_copy(
                src_ref=pbuf.at[pl.ds(c * mc + j * ms, ms),
                                pl.ds(col0, nh)],
                dst_ref=buf.at[s, pl.ds(j * ms, ms), :],
                send_sem=ssem.at[s * N_SUB + j],
                recv_sem=rsem.at[s * N_SUB + j],
                device_id=(tgt,), device_id_type=pl.DeviceIdType.MESH,
            )
            d.start()
            return d

        def ag_send(dirn, s, j):
            ssem = ag_s_cw if dirn == 0 else ag_s_ccw
            rsem = ag_r_cw if dirn == 0 else ag_r_ccw
            tgt = right if dirn == 0 else left
            src = (obuf.at[dirn, pl.ds(j * ms, ms), :] if s == 0
                   else agr.at[dirn, s - 1, pl.ds(j * ms, ms), :])
            d = pltpu.make_async_remote_copy(
                src_ref=src,
                dst_ref=agr.at[dirn, s, pl.ds(j * ms, ms), :],
                send_sem=ssem.at[s * N_SUB + j],
                recv_sem=rsem.at[s * N_SUB + j],
                device_id=(tgt,), device_id_type=pl.DeviceIdType.MESH,
            )
            d.start()
            return d

        out_cps = []

        def out_store(src, c, col0):
            cp = pltpu.make_async_copy(
                src, out_ref.at[pl.ds(c * mc, mc), pl.ds(col0, nh)],
                out_sems.at[len(out_cps)],
            )
            cp.start()
            out_cps.append(cp)

        cw = {}
        ccw = {}
        b_cps[0].wait()
        bbf[:, pl.ds(0, nh)] = bstage[:, pl.ds(0, nh)].astype(jnp.bfloat16)
        a_cps[0].wait()
        compute_half(my, 0)
        for j in range(N_SUB):
            cw[(0, j)] = rs_send(0, 0, j, my)
        b_cps[1].wait()
        bbf[:, pl.ds(nh, nh)] = bstage[:, pl.ds(nh, nh)].astype(
            jnp.bfloat16)
        compute_half(my, 1)
        for j in range(N_SUB):
            ccw[(0, j)] = rs_send(1, 0, j, my)

        a_cps[1].wait()
        compute_half(c_cw1, 0)
        a_cps[2].wait()
        compute_half(c_ccw1, 1)

        for s in (1, 2):
            c_cw = lax.rem(my - s + N_DEV, N_DEV)
            c_ccw = lax.rem(my + s, N_DEV)
            for j in range(N_SUB):
                cw[(s - 1, j)].wait_recv()
                row = pl.ds(c_cw * mc + j * ms, ms)
                pbuf[row, pl.ds(0, nh)] = (
                    rs_cw[s - 1, pl.ds(j * ms, ms), :]
                    + pbuf[row, pl.ds(0, nh)])
                cw[(s, j)] = rs_send(0, s, j, c_cw)

                ccw[(s - 1, j)].wait_recv()
                row = pl.ds(c_ccw * mc + j * ms, ms)
                pbuf[row, pl.ds(nh, nh)] = (
                    rs_ccw[s - 1, pl.ds(j * ms, ms), :]
                    + pbuf[row, pl.ds(nh, nh)])
                ccw[(s, j)] = rs_send(1, s, j, c_ccw)
            if s == 1:
                a_cps[3].wait()
                compute_half(c2, 0)
                compute_half(c2, 1)

        o_cw = lax.rem(my + 1, N_DEV)
        o_ccw = lax.rem(my + 3, N_DEV)
        compute_half(o_cw, 0)
        compute_half(o_ccw, 1)

        ag_cw = {}
        ag_ccw = {}

        for j in range(N_SUB):
            cw[(2, j)].wait_recv()
            z = (rs_cw[2, pl.ds(j * ms, ms), :].astype(jnp.float32)
                 + pbuf[pl.ds(o_cw * mc + j * ms, ms),
                        pl.ds(0, nh)].astype(jnp.float32))
            obuf[0, pl.ds(j * ms, ms), :] = (
                z * jax.nn.sigmoid(z)).astype(jnp.bfloat16)
            ag_cw[(0, j)] = ag_send(0, 0, j)
        out_store(obuf.at[0], o_cw, 0)

        for j in range(N_SUB):
            ccw[(2, j)].wait_recv()
            z = (rs_ccw[2, pl.ds(j * ms, ms), :].astype(jnp.float32)
                 + pbuf[pl.ds(o_ccw * mc + j * ms, ms),
                        pl.ds(nh, nh)].astype(jnp.float32))
            obuf[1, pl.ds(j * ms, ms), :] = (
                z * jax.nn.sigmoid(z)).astype(jnp.bfloat16)
            ag_ccw[(0, j)] = ag_send(1, 0, j)
        out_store(obuf.at[1], o_ccw, nh)

        for s in (1, 2):
            for j in range(N_SUB):
                ag_cw[(s - 1, j)].wait_recv()
                ag_cw[(s, j)] = ag_send(0, s, j)
                ag_ccw[(s - 1, j)].wait_recv()
                ag_ccw[(s, j)] = ag_send(1, s, j)
            out_store(agr.at[0, s - 1],
                      lax.rem(my - (s - 1) + N_DEV, N_DEV), 0)
            out_store(agr.at[1, s - 1], lax.rem(my + (s - 1), N_DEV), nh)
        for j in range(N_SUB):
            ag_cw[(2, j)].wait_recv()
            ag_ccw[(2, j)].wait_recv()
        out_store(agr.at[0, 2], lax.rem(my - 2 + N_DEV, N_DEV), 0)
        out_store(agr.at[1, 2], lax.rem(my + 2, N_DEV), nh)

        for cp in out_cps:
            cp.wait()
        for d in (list(cw.values()) + list(ccw.values())
                  + list(ag_cw.values()) + list(ag_ccw.values())):
            d.wait_send()

    n_sem = (N_DEV - 1) * N_SUB
    return pl.pallas_call(
        kernel_body,
        out_shape=jax.ShapeDtypeStruct((m, n), jnp.bfloat16),
        in_specs=[
            pl.BlockSpec(memory_space=pl.ANY),
            pl.BlockSpec(memory_space=pl.ANY),
        ],
        out_specs=pl.BlockSpec(memory_space=pl.ANY),
        scratch_shapes=[
            pltpu.VMEM((m, k), jnp.float32),
            pltpu.VMEM((k, n), jnp.float32),
            pltpu.VMEM((m, n), jnp.bfloat16),
            pltpu.VMEM((N_DEV - 1, mc, nh), jnp.bfloat16),
            pltpu.VMEM((N_DEV - 1, mc, nh), jnp.bfloat16),
            pltpu.VMEM((k, n), jnp.bfloat16),
            pltpu.VMEM((2, mc, nh), jnp.bfloat16),
            pltpu.VMEM((2, N_DEV - 1, mc, nh), jnp.bfloat16),
            pltpu.SemaphoreType.DMA((7,)),
            pltpu.SemaphoreType.DMA((8,)),
            pltpu.SemaphoreType.DMA((n_sem,)),
            pltpu.SemaphoreType.DMA((n_sem,)),
            pltpu.SemaphoreType.DMA((n_sem,)),
            pltpu.SemaphoreType.DMA((n_sem,)),
            pltpu.SemaphoreType.DMA((n_sem,)),
            pltpu.SemaphoreType.DMA((n_sem,)),
            pltpu.SemaphoreType.DMA((n_sem,)),
            pltpu.SemaphoreType.DMA((n_sem,)),
        ],
        compiler_params=pltpu.CompilerParams(
            collective_id=0,
            vmem_limit_bytes=128 * 1024 * 1024,
        ),
    )(A, B)
